# baseline (speedup 1.0000x reference)
import functools

import numpy as np
import jax
import jax.numpy as jnp
from jax import lax
from jax.experimental import pallas as pl
from jax.experimental.pallas import tpu as pltpu
from jax.experimental.pallas import tpu_sc as plsc

N_NODES = 10000
N_EDGES = 160000
D_IN = 4
D_MIX = 16
W_NUMEL = 256
FC0, FC1 = 16, 32
NUM_NEIGHBORS = 16.0

_z = np.random.RandomState(0).randn(1000000).astype(np.float32)
_SILU_CST = float(np.mean((_z / (1.0 + np.exp(-_z))) ** 2) ** -0.5)
del _z

_OUT_SCALE = 1.0 / (np.sqrt(float(D_IN * D_MIX)) * np.sqrt(NUM_NEIGHBORS))

_MSG_W = 8
_k = np.arange(256)
_v = np.arange(16)
_A1 = (_v[None, :] // 4 == np.arange(4)[:, None]).astype(np.float32)
_A2 = (_v[None, :] % 4 == np.arange(4)[:, None]).astype(np.float32)
_A3 = (_k[None, :] // 64 == np.arange(4)[:, None]).astype(np.float32)
_A4 = ((_k[None, :] // 4) % 16 == np.arange(16)[:, None]).astype(np.float32)
_S = (_k[:, None] % 4 == np.arange(_MSG_W)[None, :]).astype(np.float32)

_NC, _NS = 2, 16
_NW = _NC * _NS
_EPW = N_EDGES // _NW
_CH = 128
_NROWS = N_EDGES // _CH
_RPW = 40


@functools.cache
def _sc_mesh():
    return plsc.VectorSubcoreMesh(core_axis_name="c", subcore_axis_name="s",
                                  num_cores=_NC, num_subcores=_NS)


_GCH = 1000
_NGCH = _EPW // _GCH
_GGRP = (_GCH + 15) // 16


def _gather_body(src_hbm, dst_hbm, node_hbm, xsd_hbm,
                 node_v, idx_s, idx_d, xsd_v):
    cid = lax.axis_index("c")
    sid = lax.axis_index("s")
    wid = sid * _NC + cid
    base = wid * _EPW
    pltpu.sync_copy(node_hbm, node_v)
    iota = lax.iota(jnp.int32, 16)

    for k in range(_NGCH):
        cbase = base + k * _GCH
        pltpu.sync_copy(src_hbm.at[pl.ds(cbase, _GCH)],
                        idx_s.at[pl.ds(0, _GCH)])
        pltpu.sync_copy(dst_hbm.at[pl.ds(cbase, _GCH)],
                        idx_d.at[pl.ds(0, _GCH)])

        def body(g, carry):
            off = g * 16
            lanes = off + iota
            mask = lanes < _GCH
            rows = jnp.where(mask, lanes, 0)
            sv = jnp.where(mask, idx_s[pl.ds(off, 16)], 0)
            dv = jnp.where(mask, idx_d[pl.ds(off, 16)], 0)
            for c in range(D_IN):
                cc = jnp.full((16,), c, jnp.int32)
                vs = plsc.load_gather(node_v, [sv, cc])
                plsc.store_scatter(xsd_v, [rows, cc], vs, mask=mask)
                vd = plsc.load_gather(node_v, [dv, cc])
                plsc.store_scatter(xsd_v, [rows, cc + D_IN], vd, mask=mask)
            return carry

        lax.fori_loop(0, _GGRP, body, 0)
        pltpu.sync_copy(xsd_v, xsd_hbm.at[pl.ds(cbase, _GCH)])


@functools.cache
def _gather():
    return pl.kernel(
        _gather_body,
        out_type=jax.ShapeDtypeStruct((N_EDGES, 2 * D_IN), jnp.float32),
        mesh=_sc_mesh(),
        scratch_types=[
            pltpu.VMEM((N_NODES, D_IN), jnp.float32),
            pltpu.VMEM((_GCH + 16, ), jnp.int32),
            pltpu.VMEM((_GCH + 16, ), jnp.int32),
            pltpu.VMEM((_GCH, 2 * D_IN), jnp.float32),
        ],
        compiler_params=pltpu.CompilerParams(needs_layout_passes=False, use_tc_tiling_on_sc=False),
    )


def _dense_body(et_ref, xsd_hbm, w1_ref, w2_ref,
                a1_ref, a2_ref, a3_ref, a4_ref, s_ref, out_hbm,
                xsd_v, msg_v, sem_in, sem_out):
    f32 = jnp.float32
    i = pl.program_id(0)
    cp_in = pltpu.make_async_copy(xsd_hbm.at[pl.ds(i * _BE, _BE)], xsd_v,
                                  sem_in)
    cp_in.start()
    et = et_ref[...]
    cp_in.wait()
    xs = xsd_v[:, :D_IN]
    xd = xsd_v[:, D_IN:]
    w1 = w1_ref[...] * (1.0 / np.sqrt(float(FC0)))
    w2 = w2_ref[...] * (_OUT_SCALE / np.sqrt(float(FC1)))
    h = jnp.dot(et, w1, preferred_element_type=f32)
    h = _SILU_CST * h * jax.nn.sigmoid(h)
    weight = jnp.dot(h, w2, preferred_element_type=f32)
    mix = (jnp.dot(xs, a1_ref[...], preferred_element_type=f32)
           * jnp.dot(xd, a2_ref[...], preferred_element_type=f32))
    g4 = (jnp.dot(xs, a3_ref[...], preferred_element_type=f32)
          * jnp.dot(mix, a4_ref[...], preferred_element_type=f32))
    msg_v[...] = jnp.dot(g4 * weight, s_ref[...], preferred_element_type=f32)
    cp_out = pltpu.make_async_copy(msg_v, out_hbm.at[pl.ds(i * _BE, _BE)],
                                   sem_out)
    cp_out.start()
    cp_out.wait()


_BE = 4000


def _dense(edge_type, xsd, fc_w1, fc_w2):
    full = lambda shape: pl.BlockSpec(shape, lambda i: (0, 0))
    return pl.pallas_call(
        _dense_body,
        grid=(N_EDGES // _BE,),
        in_specs=[
            pl.BlockSpec((_BE, FC0), lambda i: (i, 0)),
            pl.BlockSpec(memory_space=pl.ANY),
            full((FC0, FC1)),
            full((FC1, W_NUMEL)),
            full((D_IN, D_MIX)),
            full((D_IN, D_MIX)),
            full((D_IN, W_NUMEL)),
            full((D_MIX, W_NUMEL)),
            full((W_NUMEL, _MSG_W)),
        ],
        out_specs=pl.BlockSpec(memory_space=pl.ANY),
        out_shape=jax.ShapeDtypeStruct((N_EDGES, _MSG_W), jnp.float32),
        scratch_shapes=[
            pltpu.VMEM((_BE, 2 * D_IN), jnp.float32),
            pltpu.VMEM((_BE, _MSG_W), jnp.float32),
            pltpu.SemaphoreType.DMA,
            pltpu.SemaphoreType.DMA,
        ],
    )(edge_type, xsd, fc_w1, fc_w2,
      jnp.asarray(_A1), jnp.asarray(_A2), jnp.asarray(_A3),
      jnp.asarray(_A4), jnp.asarray(_S))


def _scatter_body(dst2d_hbm, msg_hbm, zeros_hbm, out_hbm,
                  msg_v, idx_v, acc_sh):
    cid = lax.axis_index("c")
    sid = lax.axis_index("s")
    wid = sid * _NC + cid
    row0 = wid * _RPW

    @pl.when(sid == 0)
    def _():
        pltpu.sync_copy(zeros_hbm, acc_sh)

    plsc.subcore_barrier()

    def add_rows(nrows):
        pltpu.sync_copy(dst2d_hbm.at[pl.ds(row0, nrows)],
                        idx_v.at[pl.ds(0, nrows)])
        pltpu.sync_copy(msg_hbm.at[pl.ds(row0 * _CH, nrows * _CH)],
                        msg_v.at[pl.ds(0, nrows * _CH)])

        def body(j, carry):
            pltpu.sync_copy(msg_v.at[pl.ds(j * _CH, _CH)],
                            acc_sh.at[idx_v.at[j]], add=True)
            return carry

        lax.fori_loop(0, nrows, body, 0)

    @pl.when(wid < _NW - 1)
    def _():
        add_rows(_RPW)

    @pl.when(wid == _NW - 1)
    def _():
        add_rows(_NROWS - (_NW - 1) * _RPW)

    plsc.subcore_barrier()

    @pl.when(sid < 10)
    def _():
        r0 = sid * (N_NODES // 10)
        pltpu.sync_copy(acc_sh.at[pl.ds(r0, N_NODES // 10)],
                        out_hbm.at[cid, pl.ds(r0, N_NODES // 10)])


@functools.cache
def _scatter():
    return pl.kernel(
        _scatter_body,
        out_type=jax.ShapeDtypeStruct((_NC, N_NODES, _MSG_W), jnp.float32),
        mesh=_sc_mesh(),
        scratch_types=[
            pltpu.VMEM((_RPW * _CH, _MSG_W), jnp.float32),
            pltpu.VMEM((_RPW, _CH), jnp.int32),
            pltpu.VMEM_SHARED((N_NODES, _MSG_W), jnp.float32),
        ],
        compiler_params=pltpu.CompilerParams(needs_layout_passes=False, use_tc_tiling_on_sc=False),
    )


def _combine_body(p_ref, out_ref):
    out_ref[...] = (p_ref[0] + p_ref[1])[:, :D_IN]


def _combine(partials):
    return pl.pallas_call(
        _combine_body,
        out_shape=jax.ShapeDtypeStruct((N_NODES, D_IN), jnp.float32),
    )(partials)


def kernel(edge_src, edge_dst, node_pos, edge_type, fc_w1, fc_w2):
    xsd = _gather()(edge_src, edge_dst, node_pos)
    msg = _dense(edge_type, xsd, fc_w1, fc_w2)
    dst2d = edge_dst.reshape(_NROWS, _CH)
    partials = _scatter()(dst2d, msg, jnp.zeros((N_NODES, _MSG_W), jnp.float32))
    return _combine(partials)


# speedup vs baseline: 1.4078x; 1.4078x over previous
import functools

import numpy as np
import jax
import jax.numpy as jnp
from jax import lax
from jax.experimental import pallas as pl
from jax.experimental.pallas import tpu as pltpu
from jax.experimental.pallas import tpu_sc as plsc

N_NODES = 10000
N_EDGES = 160000
D_IN = 4
D_MIX = 16
W_NUMEL = 256
FC0, FC1 = 16, 32
NUM_NEIGHBORS = 16.0

_z = np.random.RandomState(0).randn(1000000).astype(np.float32)
_SILU_CST = float(np.mean((_z / (1.0 + np.exp(-_z))) ** 2) ** -0.5)
del _z

_OUT_SCALE = 1.0 / (np.sqrt(float(D_IN * D_MIX)) * np.sqrt(NUM_NEIGHBORS))

_MSG_W = 8
_k = np.arange(256)
_v = np.arange(16)
_A1 = (_v[None, :] // 4 == np.arange(4)[:, None]).astype(np.float32)
_A2 = (_v[None, :] % 4 == np.arange(4)[:, None]).astype(np.float32)
_A3 = (_k[None, :] // 64 == np.arange(4)[:, None]).astype(np.float32)
_A4 = ((_k[None, :] // 4) % 16 == np.arange(16)[:, None]).astype(np.float32)
_S = (_k[:, None] % 4 == np.arange(_MSG_W)[None, :]).astype(np.float32)

_NC, _NS = 2, 16
_NW = _NC * _NS
_EPW = N_EDGES // _NW
_CH = 128
_NROWS = N_EDGES // _CH
_RPW = 40


@functools.cache
def _sc_mesh():
    return plsc.VectorSubcoreMesh(core_axis_name="c", subcore_axis_name="s",
                                  num_cores=_NC, num_subcores=_NS)


_GCH = 1000
_NGCH = _EPW // _GCH
_GGRP = (_GCH + 15) // 16


def _gather_body(src_hbm, dst_hbm, node_hbm, xsd_hbm,
                 node_v, idx_s, idx_d, xsd_v):
    cid = lax.axis_index("c")
    sid = lax.axis_index("s")
    wid = sid * _NC + cid
    base = wid * _EPW
    pltpu.sync_copy(node_hbm, node_v)
    iota = lax.iota(jnp.int32, 16)

    for k in range(_NGCH):
        cbase = base + k * _GCH
        pltpu.sync_copy(src_hbm.at[pl.ds(cbase, _GCH)],
                        idx_s.at[pl.ds(0, _GCH)])
        pltpu.sync_copy(dst_hbm.at[pl.ds(cbase, _GCH)],
                        idx_d.at[pl.ds(0, _GCH)])

        def body(g, carry):
            off = g * 16
            lanes = off + iota
            mask = lanes < _GCH
            rows = jnp.where(mask, lanes, 0)
            sv = jnp.where(mask, idx_s[pl.ds(off, 16)], 0)
            dv = jnp.where(mask, idx_d[pl.ds(off, 16)], 0)
            for c in range(D_IN):
                cc = jnp.full((16,), c, jnp.int32)
                vs = plsc.load_gather(node_v, [sv, cc])
                plsc.store_scatter(xsd_v, [rows, cc], vs, mask=mask)
                vd = plsc.load_gather(node_v, [dv, cc])
                plsc.store_scatter(xsd_v, [rows, cc + D_IN], vd, mask=mask)
            return carry

        lax.fori_loop(0, _GGRP, body, 0)
        pltpu.sync_copy(xsd_v, xsd_hbm.at[pl.ds(cbase, _GCH)])


@functools.cache
def _gather():
    return pl.kernel(
        _gather_body,
        out_type=jax.ShapeDtypeStruct((N_EDGES, 2 * D_IN), jnp.float32),
        mesh=_sc_mesh(),
        scratch_types=[
            pltpu.VMEM((N_NODES, D_IN), jnp.float32),
            pltpu.VMEM((_GCH + 16, ), jnp.int32),
            pltpu.VMEM((_GCH + 16, ), jnp.int32),
            pltpu.VMEM((_GCH, 2 * D_IN), jnp.float32),
        ],
        compiler_params=pltpu.CompilerParams(needs_layout_passes=False, use_tc_tiling_on_sc=False),
    )


def _dense_body(et_ref, xsd_ref, w1_ref, w2_ref,
                a1_ref, a2_ref, a3_ref, a4_ref, s_ref, out_ref):
    f32 = jnp.float32
    et = et_ref[...]
    xs = xsd_ref[:, :D_IN]
    xd = xsd_ref[:, D_IN:]
    w1 = w1_ref[...] * (1.0 / np.sqrt(float(FC0)))
    w2 = w2_ref[...] * (_OUT_SCALE / np.sqrt(float(FC1)))
    h = jnp.dot(et, w1, preferred_element_type=f32)
    h = _SILU_CST * h * jax.nn.sigmoid(h)
    weight = jnp.dot(h, w2, preferred_element_type=f32)
    mix = (jnp.dot(xs, a1_ref[...], preferred_element_type=f32)
           * jnp.dot(xd, a2_ref[...], preferred_element_type=f32))
    g4 = (jnp.dot(xs, a3_ref[...], preferred_element_type=f32)
          * jnp.dot(mix, a4_ref[...], preferred_element_type=f32))
    out_ref[...] = jnp.dot(g4 * weight, s_ref[...], preferred_element_type=f32)


_BE = 8000


def _dense(edge_type, xsd, fc_w1, fc_w2):
    full = lambda shape: pl.BlockSpec(shape, lambda i: (0, 0))
    return pl.pallas_call(
        _dense_body,
        grid=(N_EDGES // _BE,),
        in_specs=[
            pl.BlockSpec((_BE, FC0), lambda i: (i, 0)),
            pl.BlockSpec((_BE, 2 * D_IN), lambda i: (i, 0)),
            full((FC0, FC1)),
            full((FC1, W_NUMEL)),
            full((D_IN, D_MIX)),
            full((D_IN, D_MIX)),
            full((D_IN, W_NUMEL)),
            full((D_MIX, W_NUMEL)),
            full((W_NUMEL, _MSG_W)),
        ],
        out_specs=pl.BlockSpec((_BE, _MSG_W), lambda i: (i, 0)),
        out_shape=jax.ShapeDtypeStruct((N_EDGES, _MSG_W), jnp.float32),
    )(edge_type, xsd, fc_w1, fc_w2,
      jnp.asarray(_A1), jnp.asarray(_A2), jnp.asarray(_A3),
      jnp.asarray(_A4), jnp.asarray(_S))


def _scatter_body(dst2d_hbm, msg_hbm, zeros_hbm, out_hbm,
                  msg_v, idx_v, acc_sh):
    cid = lax.axis_index("c")
    sid = lax.axis_index("s")
    wid = sid * _NC + cid
    row0 = wid * _RPW

    @pl.when(sid == 0)
    def _():
        pltpu.sync_copy(zeros_hbm, acc_sh)

    plsc.subcore_barrier()

    def add_rows(nrows):
        pltpu.sync_copy(dst2d_hbm.at[pl.ds(row0, nrows)],
                        idx_v.at[pl.ds(0, nrows)])
        pltpu.sync_copy(msg_hbm.at[pl.ds(row0 * _CH, nrows * _CH)],
                        msg_v.at[pl.ds(0, nrows * _CH)])

        def body(j, carry):
            pltpu.sync_copy(msg_v.at[pl.ds(j * _CH, _CH)],
                            acc_sh.at[idx_v.at[j]], add=True)
            return carry

        lax.fori_loop(0, nrows, body, 0)

    @pl.when(wid < _NW - 1)
    def _():
        add_rows(_RPW)

    @pl.when(wid == _NW - 1)
    def _():
        add_rows(_NROWS - (_NW - 1) * _RPW)

    plsc.subcore_barrier()

    @pl.when(sid < 10)
    def _():
        r0 = sid * (N_NODES // 10)
        pltpu.sync_copy(acc_sh.at[pl.ds(r0, N_NODES // 10)],
                        out_hbm.at[cid, pl.ds(r0, N_NODES // 10)])


@functools.cache
def _scatter():
    return pl.kernel(
        _scatter_body,
        out_type=jax.ShapeDtypeStruct((_NC, N_NODES, _MSG_W), jnp.float32),
        mesh=_sc_mesh(),
        scratch_types=[
            pltpu.VMEM((_RPW * _CH, _MSG_W), jnp.float32),
            pltpu.VMEM((_RPW, _CH), jnp.int32),
            pltpu.VMEM_SHARED((N_NODES, _MSG_W), jnp.float32),
        ],
        compiler_params=pltpu.CompilerParams(needs_layout_passes=False, use_tc_tiling_on_sc=False),
    )


def _combine_body(p_ref, out_ref):
    out_ref[...] = (p_ref[0] + p_ref[1])[:, :D_IN]


def _combine(partials):
    return pl.pallas_call(
        _combine_body,
        out_shape=jax.ShapeDtypeStruct((N_NODES, D_IN), jnp.float32),
    )(partials)


def kernel(edge_src, edge_dst, node_pos, edge_type, fc_w1, fc_w2):
    xsd = _gather()(edge_src, edge_dst, node_pos)
    msg = _dense(edge_type, xsd, fc_w1, fc_w2)
    dst2d = edge_dst.reshape(_NROWS, _CH)
    partials = _scatter()(dst2d, msg, jnp.zeros((N_NODES, _MSG_W), jnp.float32))
    return _combine(partials)
